# Initial kernel scaffold; baseline (speedup 1.0000x reference)
import jax
import jax.numpy as jnp
from jax.experimental import pallas as pl


def kernel(query, codebook):
    raise NotImplementedError("write your pallas kernel here")


# speedup vs baseline: 2.7854x; 2.7854x over previous
import functools

import jax
import jax.numpy as jnp
from jax.experimental import pallas as pl
from jax.experimental.pallas import tpu as pltpu

_CL = 8
_V = 1024
_DZ = 256
_B = 4096
_R = _B * _CL
_RB = 1024
_GRID = _R // _RB
_EPS = 1e-06


def _vq_block(x_ref, cb_ref, emb_ref, msg_ref, rep_ref, stats_ref, acc_ref):
    i = pl.program_id(0)

    @pl.when(i == 0)
    def _init():
        acc_ref[...] = jnp.zeros_like(acc_ref)

    x = x_ref[...]
    cb = cb_ref[...]
    lat2 = jnp.sum(x * x, axis=1, keepdims=True)
    cb2 = jnp.sum(cb * cb, axis=1)
    cross = jax.lax.dot_general(
        x, cb, (((1,), (1,)), ((), ())),
        preferred_element_type=jnp.float32)
    scores = -0.5 * (lat2 - 2.0 * cross + cb2[None, :])
    m = jnp.max(scores, axis=1, keepdims=True)
    e = jnp.exp(scores - m)
    s = jnp.sum(e, axis=1, keepdims=True)
    y = e / s

    ymax = jnp.max(y, axis=1, keepdims=True)
    col = jax.lax.broadcasted_iota(jnp.int32, (_RB, _V), 1)
    cand = jnp.where(y == ymax, col, _V)
    code = jnp.min(cand, axis=1)
    msg_ref[...] = code.reshape(_RB // _CL, _CL)

    y_hard = (col == code[:, None]).astype(jnp.float32)
    rep_ref[...] = y_hard.reshape(_RB // _CL, _CL * _V)
    quant = jax.lax.dot_general(
        y_hard, cb, (((1,), (0,)), ((), ())),
        preferred_element_type=jnp.float32)
    emb_ref[...] = quant.reshape(_RB // _CL, _CL * _DZ)

    acc_ref[0, :] += jnp.sum(y, axis=0)
    acc_ref[1, :] += jnp.sum(y * (jnp.log2(y + _EPS)), axis=0)

    @pl.when(i == _GRID - 1)
    def _finish():
        py = acc_ref[0, :] * (1.0 / _R)
        hy = -jnp.sum(py * jnp.log2(py + _EPS))
        hyx = -jnp.sum(acc_ref[1, :]) * (1.0 / _R)
        lane = jax.lax.broadcasted_iota(jnp.int32, (8, 128), 1)
        stats_ref[...] = jnp.where(lane == 0, hy, hyx)


@functools.partial(jax.jit, static_argnames=())
def _vq_call(x, codebook):
    return pl.pallas_call(
        _vq_block,
        grid=(_GRID,),
        in_specs=[
            pl.BlockSpec((_RB, _DZ), lambda i: (i, 0)),
            pl.BlockSpec((_V, _DZ), lambda i: (0, 0)),
        ],
        out_specs=[
            pl.BlockSpec((_RB // _CL, _CL * _DZ), lambda i: (i, 0)),
            pl.BlockSpec((_RB // _CL, _CL), lambda i: (i, 0)),
            pl.BlockSpec((_RB // _CL, _CL * _V), lambda i: (i, 0)),
            pl.BlockSpec((8, 128), lambda i: (0, 0)),
        ],
        out_shape=[
            jax.ShapeDtypeStruct((_B, _CL * _DZ), jnp.float32),
            jax.ShapeDtypeStruct((_B, _CL), jnp.int32),
            jax.ShapeDtypeStruct((_B, _CL * _V), jnp.float32),
            jax.ShapeDtypeStruct((8, 128), jnp.float32),
        ],
        scratch_shapes=[pltpu.VMEM((8, _V), jnp.float32)],
    )(x, codebook)


def kernel(query, codebook):
    x = query.reshape(_R, _DZ)
    emb, msg, rep, stats = _vq_call(x, codebook)
    latent = query.reshape(_B, _CL, _DZ)
    hy = stats[0, 0]
    hyx = stats[0, 1]
    loss = jnp.float32(0.0)
    return (latent, emb, msg, rep, hy, hyx, loss)
